# baseline (speedup 1.0000x reference)
import jax
import jax.numpy as jnp
from jax import lax
from jax.experimental import pallas as pl
from jax.experimental.pallas import tpu as pltpu
from jax.experimental.pallas import tpu_sc as plsc

NUM_ROWS = 100000
D = 128
B = 1024
S = 50
T = 5

NC = 2
NS = 16
NW = NC * NS
R = B * S
R_W = R // NW
GI = 80
CH = 2
GROWS = CH * GI // T
NG = R_W // GROWS
NP = NG // 2
LANES = D // 16


def _sc_body(idx_hbm, table_hbm, out_hbm, idx_v, stg_v, outb_v,
             gsem0, gsem1, wsem0, wsem1):
    wid = lax.axis_index("s") * NC + lax.axis_index("c")
    base = wid * R_W
    gsems = (gsem0, gsem1)
    wsems = (wsem0, wsem1)

    pltpu.sync_copy(idx_hbm.at[wid], idx_v)

    def fire(g, buf):
        for c in range(CH):
            pltpu.async_copy(table_hbm.at[idx_v.at[g, c]],
                             stg_v.at[buf, pl.ds(c * GI, GI)], gsems[buf])

    def drain(g, buf):
        for c in range(CH):
            pltpu.make_async_copy(table_hbm.at[idx_v.at[g, c]],
                                  stg_v.at[buf, pl.ds(c * GI, GI)],
                                  gsems[buf]).wait()

    def wb_wait(buf):
        pltpu.make_async_copy(outb_v.at[buf], out_hbm.at[pl.ds(base, GROWS)],
                              wsems[buf]).wait()

    def compute(buf):
        def row_sum(r, carry):
            q = r * T
            for c in range(LANES):
                col = pl.ds(c * 16, 16)
                acc = stg_v[buf, q, col]
                for j in range(1, T):
                    acc = acc + stg_v[buf, q + j, col]
                outb_v[buf, r, col] = acc
            return carry

        lax.fori_loop(0, GROWS, row_sum, 0, unroll=2)

    def wb_write(g, buf):
        pltpu.async_copy(outb_v.at[buf],
                         out_hbm.at[pl.ds(base + g * GROWS, GROWS)],
                         wsems[buf])

    def half(g, buf, do_wb_wait, fire_next):
        drain(g, buf)
        if do_wb_wait:
            wb_wait(buf)
        compute(buf)
        if fire_next:
            fire(g + 2, buf)
        wb_write(g, buf)

    fire(0, 0)
    fire(1, 1)
    half(0, 0, False, True)
    half(1, 1, False, True)

    def pair(p, carry):
        g0 = p * 2
        half(g0, 0, True, True)
        half(g0 + 1, 1, True, True)
        return carry

    lax.fori_loop(1, NP - 1, pair, 0)

    half(NG - 2, 0, True, False)
    half(NG - 1, 1, True, False)

    wb_wait(0)
    wb_wait(1)


def kernel(morphemes, table):
    idx = morphemes.astype(jnp.int32).reshape(NW, NG, CH, GI)

    sc_kernel = pl.kernel(
        _sc_body,
        out_type=jax.ShapeDtypeStruct((R, D), jnp.float32),
        mesh=plsc.VectorSubcoreMesh(core_axis_name="c", subcore_axis_name="s"),
        scratch_types=[
            pltpu.VMEM((NG, CH, GI), jnp.int32),
            pltpu.VMEM((2, CH * GI, D), jnp.float32),
            pltpu.VMEM((2, GROWS, D), jnp.float32),
            pltpu.SemaphoreType.DMA,
            pltpu.SemaphoreType.DMA,
            pltpu.SemaphoreType.DMA,
            pltpu.SemaphoreType.DMA,
        ],
    )
    out = sc_kernel(idx, table)
    return out.reshape(B, S, D)
